# baseline (speedup 1.0000x reference)
import jax
import jax.numpy as jnp
from jax.experimental import pallas as pl
from jax.experimental.pallas import tpu as pltpu

S = 7
C = 20
NBOX = 2
CELL = C + NBOX * 5
BATCH = 8
MID = 2048
IN_F = 1024 * S * S
OUT_F = S * S * CELL
K_BLK = 512
K_TILES = IN_F // K_BLK


def _stream_kernel(x_ref, w1_ref, b1_ref, h_ref):
    k = pl.program_id(0)

    @pl.when(k == 0)
    def _init():
        h_ref[...] = jnp.broadcast_to(b1_ref[...], h_ref.shape)

    h_ref[...] += jnp.dot(
        x_ref[...], w1_ref[...], preferred_element_type=jnp.float32
    )


def _head_kernel(h_ref, w2t_ref, b2_ref, out_ref):
    h = h_ref[...]
    h = jnp.where(h > 0, h, 0.1 * h)
    o = jax.lax.dot_general(
        h, w2t_ref[...],
        dimension_numbers=(((1,), (1,)), ((), ())),
        preferred_element_type=jnp.float32,
    )
    o = o + b2_ref[...]
    col = jax.lax.broadcasted_iota(jnp.int32, o.shape, 1)
    r = col % CELL
    m = (r < C) & ((r % 5 == 1) | (r % 5 == 2))
    out_ref[...] = jnp.where(m, jax.nn.sigmoid(o), o)


def kernel(x, W1, b1, W2, b2):
    x2 = x.reshape(BATCH, IN_F)
    h = pl.pallas_call(
        _stream_kernel,
        grid=(K_TILES,),
        in_specs=[
            pl.BlockSpec((BATCH, K_BLK), lambda k: (0, k)),
            pl.BlockSpec((K_BLK, MID), lambda k: (k, 0)),
            pl.BlockSpec((1, MID), lambda k: (0, 0)),
        ],
        out_specs=pl.BlockSpec((BATCH, MID), lambda k: (0, 0)),
        out_shape=jax.ShapeDtypeStruct((BATCH, MID), jnp.float32),
        compiler_params=pltpu.CompilerParams(
            dimension_semantics=("arbitrary",),
        ),
    )(x2, W1, b1[None, :])
    out = pl.pallas_call(
        _head_kernel,
        out_shape=jax.ShapeDtypeStruct((BATCH, OUT_F), jnp.float32),
    )(h, W2.T, b2[None, :])
    return out.reshape(-1, S, S, CELL)


# speedup vs baseline: 1.0890x; 1.0890x over previous
import jax
import jax.numpy as jnp
from jax.experimental import pallas as pl
from jax.experimental.pallas import tpu as pltpu

S = 7
C = 20
NBOX = 2
CELL = C + NBOX * 5
BATCH = 8
MID = 2048
IN_F = 1024 * S * S
OUT_F = S * S * CELL
K_BLK = 1024
K_TILES = IN_F // K_BLK


def _stream_kernel(x_ref, w1_ref, b1_ref, h_ref):
    k = pl.program_id(0)

    @pl.when(k == 0)
    def _init():
        h_ref[...] = jnp.broadcast_to(b1_ref[...], h_ref.shape)

    h_ref[...] += jnp.dot(
        x_ref[...], w1_ref[...], preferred_element_type=jnp.float32
    )


def _head_kernel(h_ref, w2t_ref, b2_ref, out_ref):
    h = h_ref[...]
    h = jnp.where(h > 0, h, 0.1 * h)
    o = jax.lax.dot_general(
        h, w2t_ref[...],
        dimension_numbers=(((1,), (1,)), ((), ())),
        preferred_element_type=jnp.float32,
    )
    o = o + b2_ref[...]
    col = jax.lax.broadcasted_iota(jnp.int32, o.shape, 1)
    r = col % CELL
    m = (r < C) & ((r % 5 == 1) | (r % 5 == 2))
    out_ref[...] = jnp.where(m, jax.nn.sigmoid(o), o)


def kernel(x, W1, b1, W2, b2):
    x2 = x.reshape(BATCH, IN_F)
    h = pl.pallas_call(
        _stream_kernel,
        grid=(K_TILES,),
        in_specs=[
            pl.BlockSpec((BATCH, K_BLK), lambda k: (0, k)),
            pl.BlockSpec((K_BLK, MID), lambda k: (k, 0)),
            pl.BlockSpec((1, MID), lambda k: (0, 0)),
        ],
        out_specs=pl.BlockSpec((BATCH, MID), lambda k: (0, 0)),
        out_shape=jax.ShapeDtypeStruct((BATCH, MID), jnp.float32),
        compiler_params=pltpu.CompilerParams(
            dimension_semantics=("arbitrary",),
        ),
    )(x2, W1, b1[None, :])
    out = pl.pallas_call(
        _head_kernel,
        out_shape=jax.ShapeDtypeStruct((BATCH, OUT_F), jnp.float32),
    )(h, W2.T, b2[None, :])
    return out.reshape(-1, S, S, CELL)
